# Initial kernel scaffold; baseline (speedup 1.0000x reference)
import jax
import jax.numpy as jnp
from jax.experimental import pallas as pl


def kernel(curr_heightmap, new_roi, mask):
    raise NotImplementedError("write your pallas kernel here")


# speedup vs baseline: 79.3664x; 79.3664x over previous
import functools

import jax
import jax.numpy as jnp
from jax import lax
from jax.experimental import pallas as pl
from jax.experimental.pallas import tpu as pltpu
from jax.experimental.pallas import tpu_sc as plsc

_NC = 2
_NS = 16
_NW = _NC * _NS
_L = 16


def _make_sc_partials(B, H, W):
    HW = H * W
    assert B % _NW == 0 and HW % _L == 0
    bpw = B // _NW
    n_steps = HW // _L
    mesh = plsc.VectorSubcoreMesh(core_axis_name="c", subcore_axis_name="s")

    @functools.partial(
        pl.kernel,
        mesh=mesh,
        compiler_params=pltpu.CompilerParams(needs_layout_passes=False),
        out_type=[
            jax.ShapeDtypeStruct((_NW, _L), jnp.float32),
            jax.ShapeDtypeStruct((_NW, _L), jnp.float32),
        ],
        scratch_types=[
            pltpu.VMEM((3, HW), jnp.float32),
            pltpu.VMEM((HW,), jnp.float32),
            pltpu.VMEM((HW,), jnp.float32),
            pltpu.VMEM((_L,), jnp.float32),
            pltpu.VMEM((_L,), jnp.float32),
        ],
    )
    def sc_kernel(curr_hbm, roi_hbm, mask_hbm, loss_out, cnt_out,
                  roi_v, curr_v, mask_v, loss_v, cnt_v):
        wid = lax.axis_index("s") * _NC + lax.axis_index("c")
        base = wid * bpw

        def batch_body(i, accs):
            b = base + i
            pltpu.sync_copy(curr_hbm.at[b], curr_v)
            pltpu.sync_copy(roi_hbm.at[b], roi_v)
            pltpu.sync_copy(mask_hbm.at[b], mask_v)

            def step(j, accs2):
                lacc, cacc = accs2
                sl = pl.ds(j * _L, _L)
                yf = roi_v[0, sl]
                xf = roi_v[1, sl]
                t = roi_v[2, sl]
                y = yf.astype(jnp.int32)
                x = xf.astype(jnp.int32)
                valid = (y >= 0) & (y < H) & (x >= 0) & (x < W)
                yc = jnp.minimum(jnp.maximum(y, 0), H - 1)
                xc = jnp.minimum(jnp.maximum(x, 0), W - 1)
                flat = yc * W + xc
                c = plsc.load_gather(curr_v, [flat])
                m = plsc.load_gather(mask_v, [flat])
                d = c - t
                ad = jnp.abs(d)
                loss = jnp.where(ad < 1.0, 0.5 * d * d, ad - 0.5)
                w = jnp.where(valid, m, 0.0)
                return (lacc + loss * w, cacc + w)

            return lax.fori_loop(0, n_steps, step, accs)

        zero = jnp.zeros((_L,), jnp.float32)
        lacc, cacc = lax.fori_loop(0, bpw, batch_body, (zero, zero))
        loss_v[...] = lacc
        cnt_v[...] = cacc
        pltpu.sync_copy(loss_v, loss_out.at[wid])
        pltpu.sync_copy(cnt_v, cnt_out.at[wid])

    return sc_kernel


def _finish(loss_ref, cnt_ref, out_ref):
    ls = jnp.sum(loss_ref[...])
    nv = jnp.sum(cnt_ref[...])
    out_ref[...] = (ls / (nv + 1e-6)).reshape(1, 1)


def kernel(curr_heightmap, new_roi, mask):
    B, _, H, W = curr_heightmap.shape
    HW = H * W
    curr2 = curr_heightmap.reshape(B, HW)
    roi2 = new_roi.reshape(B, 3, HW)
    mask2 = mask.reshape(B, HW)
    loss_p, cnt_p = _make_sc_partials(B, H, W)(curr2, roi2, mask2)
    out = pl.pallas_call(
        _finish,
        out_shape=jax.ShapeDtypeStruct((1, 1), jnp.float32),
    )(loss_p, cnt_p)
    return out[0, 0]


# speedup vs baseline: 83.7490x; 1.0552x over previous
import functools

import jax
import jax.numpy as jnp
from jax import lax
from jax.experimental import pallas as pl
from jax.experimental.pallas import tpu as pltpu
from jax.experimental.pallas import tpu_sc as plsc

_NC = 2
_NS = 16
_NW = _NC * _NS
_L = 16


def _make_sc_partials(B, H, W):
    HW = H * W
    assert B % _NW == 0 and HW % _L == 0
    bpw = B // _NW
    n_steps = HW // _L
    mesh = plsc.VectorSubcoreMesh(core_axis_name="c", subcore_axis_name="s")

    @functools.partial(
        pl.kernel,
        mesh=mesh,
        compiler_params=pltpu.CompilerParams(needs_layout_passes=False),
        out_type=[
            jax.ShapeDtypeStruct((_NW, _L), jnp.float32),
            jax.ShapeDtypeStruct((_NW, _L), jnp.float32),
        ],
        scratch_types=[
            pltpu.VMEM((3, HW), jnp.float32),
            pltpu.VMEM((HW,), jnp.float32),
            pltpu.VMEM((HW,), jnp.float32),
            pltpu.VMEM((_L,), jnp.float32),
            pltpu.VMEM((_L,), jnp.float32),
        ],
    )
    def sc_kernel(curr_hbm, roi_hbm, mask_hbm, loss_out, cnt_out,
                  roi_v, curr_v, mask_v, loss_v, cnt_v):
        wid = lax.axis_index("s") * _NC + lax.axis_index("c")
        base = wid * bpw

        def batch_body(i, accs):
            b = base + i
            pltpu.sync_copy(curr_hbm.at[b], curr_v)
            pltpu.sync_copy(roi_hbm.at[b], roi_v)
            pltpu.sync_copy(mask_hbm.at[b], mask_v)

            def step(j, accs2):
                lacc, cacc = accs2
                sl = pl.ds(j * _L, _L)
                yf = roi_v[0, sl]
                xf = roi_v[1, sl]
                t = roi_v[2, sl]
                y = yf.astype(jnp.int32)
                x = xf.astype(jnp.int32)
                valid = (lax.bitcast_convert_type(y, jnp.uint32) < H) & (
                    lax.bitcast_convert_type(x, jnp.uint32) < W)
                flat = jnp.minimum(jnp.maximum(y * W + x, 0), H * W - 1)
                c = plsc.load_gather(curr_v, [flat])
                m = plsc.load_gather(mask_v, [flat])
                d = c - t
                ad = jnp.abs(d)
                loss = jnp.where(ad < 1.0, 0.5 * d * d, ad - 0.5)
                w = jnp.where(valid, m, 0.0)
                return (lacc + loss * w, cacc + w)

            return lax.fori_loop(0, n_steps, step, accs, unroll=8)

        zero = jnp.zeros((_L,), jnp.float32)
        lacc, cacc = lax.fori_loop(0, bpw, batch_body, (zero, zero))
        loss_v[...] = lacc
        cnt_v[...] = cacc
        pltpu.sync_copy(loss_v, loss_out.at[wid])
        pltpu.sync_copy(cnt_v, cnt_out.at[wid])

    return sc_kernel


def _finish(loss_ref, cnt_ref, out_ref):
    ls = jnp.sum(loss_ref[...])
    nv = jnp.sum(cnt_ref[...])
    out_ref[...] = (ls / (nv + 1e-6)).reshape(1, 1)


def kernel(curr_heightmap, new_roi, mask):
    B, _, H, W = curr_heightmap.shape
    HW = H * W
    curr2 = curr_heightmap.reshape(B, HW)
    roi2 = new_roi.reshape(B, 3, HW)
    mask2 = mask.reshape(B, HW)
    loss_p, cnt_p = _make_sc_partials(B, H, W)(curr2, roi2, mask2)
    out = pl.pallas_call(
        _finish,
        out_shape=jax.ShapeDtypeStruct((1, 1), jnp.float32),
    )(loss_p, cnt_p)
    return out[0, 0]


# speedup vs baseline: 110.1578x; 1.3153x over previous
import functools

import jax
import jax.numpy as jnp
from jax import lax
from jax.experimental import pallas as pl
from jax.experimental.pallas import tpu as pltpu
from jax.experimental.pallas import tpu_sc as plsc

_NC = 2
_NS = 16
_NW = _NC * _NS
_L = 16


def _make_sc_partials(B, H, W):
    HW = H * W
    assert B % (2 * _NW) == 0 and HW % _L == 0
    bpw = B // _NW
    n_steps = HW // _L
    mesh = plsc.VectorSubcoreMesh(core_axis_name="c", subcore_axis_name="s")

    @functools.partial(
        pl.kernel,
        mesh=mesh,
        compiler_params=pltpu.CompilerParams(needs_layout_passes=False),
        out_type=[
            jax.ShapeDtypeStruct((_NW, _L), jnp.float32),
            jax.ShapeDtypeStruct((_NW, _L), jnp.float32),
        ],
        scratch_types=[
            pltpu.VMEM((3, HW), jnp.float32),
            pltpu.VMEM((3, HW), jnp.float32),
            pltpu.VMEM((HW,), jnp.float32),
            pltpu.VMEM((HW,), jnp.float32),
            pltpu.VMEM((HW,), jnp.float32),
            pltpu.VMEM((HW,), jnp.float32),
            pltpu.VMEM((_L,), jnp.float32),
            pltpu.VMEM((_L,), jnp.float32),
            pltpu.SemaphoreType.DMA,
            pltpu.SemaphoreType.DMA,
        ],
    )
    def sc_kernel(curr_hbm, roi_hbm, mask_hbm, loss_out, cnt_out,
                  roi_v0, roi_v1, curr_v0, curr_v1, mask_v0, mask_v1,
                  loss_v, cnt_v, sem0, sem1):
        wid = lax.axis_index("s") * _NC + lax.axis_index("c")
        base = wid * bpw
        bufs = ((roi_v0, curr_v0, mask_v0, sem0),
                (roi_v1, curr_v1, mask_v1, sem1))

        def fire(b, k):
            roi_v, curr_v, mask_v, sem = bufs[k]
            pltpu.make_async_copy(curr_hbm.at[b], curr_v, sem).start()
            pltpu.make_async_copy(roi_hbm.at[b], roi_v, sem).start()
            pltpu.make_async_copy(mask_hbm.at[b], mask_v, sem).start()

        def drain(b, k):
            roi_v, curr_v, mask_v, sem = bufs[k]
            pltpu.make_async_copy(curr_hbm.at[b], curr_v, sem).wait()
            pltpu.make_async_copy(roi_hbm.at[b], roi_v, sem).wait()
            pltpu.make_async_copy(mask_hbm.at[b], mask_v, sem).wait()

        def compute(k, accs):
            roi_v, curr_v, mask_v, _ = bufs[k]

            def step(j, accs2):
                lacc, cacc = accs2
                sl = pl.ds(j * _L, _L)
                yf = roi_v[0, sl]
                xf = roi_v[1, sl]
                t = roi_v[2, sl]
                y = yf.astype(jnp.int32)
                x = xf.astype(jnp.int32)
                valid = (lax.bitcast_convert_type(y, jnp.uint32) < H) & (
                    lax.bitcast_convert_type(x, jnp.uint32) < W)
                flat = jnp.minimum(jnp.maximum(y * W + x, 0), HW - 1)
                c = plsc.load_gather(curr_v, [flat])
                m = plsc.load_gather(mask_v, [flat])
                d = c - t
                ad = jnp.abs(d)
                loss = jnp.where(ad < 1.0, 0.5 * d * d, ad - 0.5)
                w = jnp.where(valid, m, 0.0)
                return (lacc + loss * w, cacc + w)

            return lax.fori_loop(0, n_steps, step, accs, unroll=8)

        fire(base, 0)

        def pair_body(ip, accs):
            for k in (0, 1):
                i = 2 * ip + k
                b = base + i

                @pl.when(i + 1 < bpw)
                def _():
                    fire(b + 1, 1 - k)

                drain(b, k)
                accs = compute(k, accs)
            return accs

        zero = jnp.zeros((_L,), jnp.float32)
        lacc, cacc = lax.fori_loop(0, bpw // 2, pair_body, (zero, zero))
        loss_v[...] = lacc
        cnt_v[...] = cacc
        pltpu.sync_copy(loss_v, loss_out.at[wid])
        pltpu.sync_copy(cnt_v, cnt_out.at[wid])

    return sc_kernel


def _finish(loss_ref, cnt_ref, out_ref):
    ls = jnp.sum(loss_ref[...])
    nv = jnp.sum(cnt_ref[...])
    out_ref[...] = (ls / (nv + 1e-6)).reshape(1, 1)


def kernel(curr_heightmap, new_roi, mask):
    B, _, H, W = curr_heightmap.shape
    HW = H * W
    curr2 = curr_heightmap.reshape(B, HW)
    roi2 = new_roi.reshape(B, 3, HW)
    mask2 = mask.reshape(B, HW)
    loss_p, cnt_p = _make_sc_partials(B, H, W)(curr2, roi2, mask2)
    out = pl.pallas_call(
        _finish,
        out_shape=jax.ShapeDtypeStruct((1, 1), jnp.float32),
    )(loss_p, cnt_p)
    return out[0, 0]
